# baseline (speedup 1.0000x reference)
import functools

import jax
import jax.numpy as jnp
from jax import lax
from jax.experimental import pallas as pl
from jax.experimental.pallas import tpu as pltpu
from jax.experimental.pallas import tpu_sc as plsc

_VOCAB = 100000
_D = 64
_MULT = 8.0

_NC = 2
_NS = 16
_NW = _NC * _NS

_C = 128
_L = 16


def _make_kernel(batch, seq):
    total = batch * seq
    assert total % (_NW * _C) == 0
    per_w = total // _NW
    groups = per_w // _C
    sgroups = batch // _C
    mesh = plsc.VectorSubcoreMesh(core_axis_name="c", subcore_axis_name="s")

    @functools.partial(
        pl.kernel,
        out_type=jax.ShapeDtypeStruct((seq * _D, batch), jnp.float32),
        mesh=mesh,
        scratch_types=(
            [pltpu.VMEM((groups, _C), jnp.int32)]
            + [pltpu.VMEM((_C, _D), jnp.float32) for _ in range(2)]
            + [pltpu.VMEM((_D, _C), jnp.float32) for _ in range(2)]
            + [pltpu.SemaphoreType.DMA for _ in range(4)]
        ),
        compiler_params=pltpu.CompilerParams(use_tc_tiling_on_sc=False, needs_layout_passes=False),
    )
    def gather_kernel(table_hbm, idx_hbm, out_hbm, idx_v,
                      gb0, gb1, tb0, tb1, gs0, gs1, os0, os1):
        gbuf = (gb0, gb1)
        tbuf = (tb0, tb1)
        gsem = (gs0, gs1)
        osem = (os0, os1)
        wid = lax.axis_index("s") * _NC + lax.axis_index("c")
        fbase = wid * groups

        pltpu.sync_copy(idx_hbm.at[wid], idx_v)

        iota16 = lax.iota(jnp.int32, _L)
        rows = [iota16 + (c * _L) for c in range(_C // _L)]

        def g_start(b, g):
            pltpu.make_async_copy(
                table_hbm.at[idx_v.at[g]], gbuf[b], gsem[b]).start()

        def g_wait(b):
            pltpu.make_async_copy(
                table_hbm.at[idx_v.at[0]], gbuf[b], gsem[b]).wait()

        def o_start(b, g):
            f = fbase + g
            s = f // sgroups
            b0 = (f % sgroups) * _C
            pltpu.make_async_copy(
                tbuf[b], out_hbm.at[pl.ds(s * _D, _D), pl.ds(b0, _C)],
                osem[b]).start()

        def o_wait(b):
            pltpu.make_async_copy(
                tbuf[b], out_hbm.at[pl.ds(0, _D), pl.ds(0, _C)],
                osem[b]).wait()

        def transpose(b):
            src, dst = gbuf[b], tbuf[b]

            def dbody(d, carry):
                cols = jnp.full((_L,), d, dtype=jnp.int32)
                for c in range(_C // _L):
                    v = plsc.load_gather(src, [rows[c], cols])
                    dst[d, pl.ds(c * _L, _L)] = v * _MULT
                return carry

            lax.fori_loop(0, _D, dbody, 0)

        g_start(0, 0)
        g_start(1, 1)
        for b in range(2):
            g_wait(b)
            transpose(b)
            o_start(b, b)
            g_start(b, b + 2)

        def step(k, carry):
            for b in range(2):
                g = 2 + 2 * k + b
                g_wait(b)
                o_wait(b)
                transpose(b)
                o_start(b, g)
                g_start(b, g + 2)
            return carry

        lax.fori_loop(0, (groups - 4) // 2, step, 0)

        for b in range(2):
            g = groups - 2 + b
            g_wait(b)
            o_wait(b)
            transpose(b)
            o_start(b, g)
        for b in range(2):
            o_wait(b)

    return gather_kernel


def kernel(input_ids, embed_weight):
    batch, seq = input_ids.shape
    total = batch * seq
    idx = input_ids.T.reshape(_NW, total // (_NW * _C), _C).astype(jnp.int32)
    ot = _make_kernel(batch, seq)(embed_weight, idx)
    return jnp.transpose(ot.reshape(seq, _D, batch), (2, 0, 1))


# speedup vs baseline: 2.0665x; 2.0665x over previous
import functools

import jax
import jax.numpy as jnp
from jax import lax
from jax.experimental import pallas as pl
from jax.experimental.pallas import tpu as pltpu
from jax.experimental.pallas import tpu_sc as plsc

_VOCAB = 100000
_D = 64
_MULT = 8.0

_NC = 2
_NS = 16
_NW = _NC * _NS

_C = 128
_NBUF = 4


def _scale_body(w_ref, o_ref):
    o_ref[...] = w_ref[...] * _MULT


def _scale_table(w):
    rows_per_block = 4000
    grid = _VOCAB // rows_per_block
    return pl.pallas_call(
        _scale_body,
        out_shape=jax.ShapeDtypeStruct((_VOCAB, _D), jnp.float32),
        grid=(grid,),
        in_specs=[pl.BlockSpec((rows_per_block, _D), lambda i: (i, 0))],
        out_specs=pl.BlockSpec((rows_per_block, _D), lambda i: (i, 0)),
    )(w)


def _make_gather(total_rows):
    assert total_rows % (_NW * _C) == 0
    per_w = total_rows // _NW
    groups = per_w // _C
    main = groups - _NBUF
    assert main % _NBUF == 0
    mesh = plsc.VectorSubcoreMesh(core_axis_name="c", subcore_axis_name="s")

    @functools.partial(
        pl.kernel,
        out_type=jax.ShapeDtypeStruct((total_rows, _D), jnp.float32),
        mesh=mesh,
        scratch_types=(
            [pltpu.VMEM((groups, _C), jnp.int32)]
            + [pltpu.VMEM((_C, _D), jnp.float32) for _ in range(_NBUF)]
            + [pltpu.SemaphoreType.DMA for _ in range(2 * _NBUF)]
        ),
        compiler_params=pltpu.CompilerParams(use_tc_tiling_on_sc=False),
    )
    def gather_kernel(table_hbm, idx_hbm, out_hbm, idx_v, *rest):
        bufs = rest[:_NBUF]
        gsem = rest[_NBUF:2 * _NBUF]
        osem = rest[2 * _NBUF:]
        wid = lax.axis_index("s") * _NC + lax.axis_index("c")
        base = wid * per_w

        pltpu.sync_copy(idx_hbm.at[wid], idx_v)

        def g_start(b, g):
            pltpu.make_async_copy(
                table_hbm.at[idx_v.at[g]], bufs[b], gsem[b]).start()

        def g_wait(b):
            pltpu.make_async_copy(
                table_hbm.at[idx_v.at[0]], bufs[b], gsem[b]).wait()

        def o_start(b, g):
            pltpu.make_async_copy(
                bufs[b], out_hbm.at[pl.ds(base + g * _C, _C)], osem[b]).start()

        def o_wait(b):
            pltpu.make_async_copy(
                bufs[b], out_hbm.at[pl.ds(base, _C)], osem[b]).wait()

        for b in range(_NBUF):
            g_start(b, b)

        def step(go, carry):
            for b in range(_NBUF):
                g = go * _NBUF + b
                g_wait(b)
                o_start(b, g)
                o_wait(b)
                g_start(b, g + _NBUF)
            return carry

        lax.fori_loop(0, main // _NBUF, step, 0)

        for b in range(_NBUF):
            g = main + b
            g_wait(b)
            o_start(b, g)
        for b in range(_NBUF):
            o_wait(b)

    return gather_kernel


def kernel(input_ids, embed_weight):
    batch, seq = input_ids.shape
    total = batch * seq
    idx = input_ids.reshape(_NW, total // (_NW * _C), _C).astype(jnp.int32)
    table = _scale_table(embed_weight)
    out = _make_gather(total)(table, idx)
    return out.reshape(batch, seq, _D)
